# baseline (speedup 1.0000x reference)
import functools

import jax
import jax.numpy as jnp
from jax import lax
from jax.experimental import pallas as pl
from jax.experimental.pallas import tpu as pltpu
from jax.experimental.pallas import tpu_sc as plsc

N = 10000
E = 320000
F = 64
H = 128
NC = 2
NS = 16
NW = NC * NS

LANE = 128
EROWS = E // LANE
RPT = EROWS // NW
REM = EROWS - RPT * NW


def _wid_rows(cid, sid):
    wid = cid * NS + sid
    base = wid * RPT + jnp.minimum(wid, REM)
    extra = wid < REM
    return base, extra


def _stage_idx(eh, buf, base, extra):
    pltpu.sync_copy(eh.at[pl.ds(base, RPT)], buf.at[pl.ds(0, RPT)])

    @pl.when(extra)
    def _():
        pltpu.sync_copy(eh.at[pl.ds(base + RPT, 1)], buf.at[pl.ds(RPT, 1)])


def _sc_degrees(e0, e1, zdeg):
    mesh = plsc.VectorSubcoreMesh(core_axis_name="c", subcore_axis_name="s")

    @functools.partial(
        pl.kernel,
        out_type=jax.ShapeDtypeStruct((NC, 4, N), jnp.float32),
        mesh=mesh,
        scratch_types=(
            [pltpu.VMEM_SHARED((N,), jnp.float32)] * 4
            + [pltpu.VMEM((RPT + 1, LANE), jnp.int32)] * 4
            + [pltpu.VMEM((LANE,), jnp.float32)]
            + [pltpu.SemaphoreType.DMA] * 4
        ),
        compiler_params=pltpu.CompilerParams(use_tc_tiling_on_sc=False),
    )
    def k(e0_h, e1_h, zdeg_h, out_h, *refs):
        accs = refs[0:4]
        idxs = refs[4:8]
        ones_v = refs[8]
        sems = refs[9:13]
        cid = lax.axis_index("c")
        sid = lax.axis_index("s")
        base, extra = _wid_rows(cid, sid)

        @pl.when(sid == 0)
        def _():
            for acc in accs:
                pltpu.sync_copy(zdeg_h, acc)

        for i in range(LANE // 16):
            ones_v[pl.ds(i * 16, 16)] = jnp.full((16,), 1.0, jnp.float32)
        for eh, buf in zip((e0_h.at[0], e0_h.at[1], e1_h.at[0], e1_h.at[1]),
                           idxs):
            _stage_idx(eh, buf, base, extra)
        plsc.subcore_barrier()

        def sstart(r, k):
            pltpu.async_copy(ones_v, accs[k].at[idxs[k].at[r]],
                             sems[k], add=True)

        def swait(r, k):
            pltpu.make_async_copy(ones_v, accs[k].at[idxs[k].at[r]],
                                  sems[k]).wait()

        for k4 in range(4):
            sstart(0, k4)

        def body(r, _):
            for k4 in range(4):
                swait(r - 1, k4)
                sstart(r, k4)
            return 0
        lax.fori_loop(1, RPT, body, 0)
        for k4 in range(4):
            swait(RPT - 1, k4)

        @pl.when(extra)
        def _():
            for k4 in range(4):
                sstart(RPT, k4)
            for k4 in range(4):
                swait(RPT, k4)
        plsc.subcore_barrier()

        @pl.when(sid < 10)
        def _():
            o = pl.multiple_of(sid * 1000, 8)
            for j, acc in enumerate(accs):
                pltpu.sync_copy(acc.at[pl.ds(o, 1000)],
                                out_h.at[cid, j, pl.ds(o, 1000)])

    return k(e0, e1, zdeg)


def _sc_messages(h0, h1, e0, e1, zagg):
    mesh = plsc.VectorSubcoreMesh(core_axis_name="c", subcore_axis_name="s")
    nbuf = 3

    @functools.partial(
        pl.kernel,
        out_type=jax.ShapeDtypeStruct((2, NC, N, F), jnp.float32),
        mesh=mesh,
        scratch_types=(
            [pltpu.VMEM_SHARED((N, F), jnp.float32),
             pltpu.VMEM_SHARED((N, F), jnp.float32),
             pltpu.VMEM((RPT + 1, LANE), jnp.int32),
             pltpu.VMEM((RPT + 1, LANE), jnp.int32)]
            + [pltpu.VMEM((LANE, F), jnp.float32)] * nbuf
            + [pltpu.SemaphoreType.DMA] * (2 * nbuf)
        ),
        compiler_params=pltpu.CompilerParams(use_tc_tiling_on_sc=False),
    )
    def k(h0_h, h1_h, e0_h, e1_h, zagg_h, out_h,
          agg, h_s, idxs_v, idxd_v, *bufs_and_sems):
        rows = bufs_and_sems[:nbuf]
        gsem = bufs_and_sems[nbuf:2 * nbuf]
        ssem = bufs_and_sems[2 * nbuf:]
        cid = lax.axis_index("c")
        sid = lax.axis_index("s")
        base, extra = _wid_rows(cid, sid)
        nrows = RPT + extra.astype(jnp.int32)

        @pl.when(sid == 0)
        def _():
            pltpu.sync_copy(zagg_h, agg)

        for g, (eh, hh) in enumerate(((e0_h, h0_h), (e1_h, h1_h))):
            sh = eh.at[0]
            dh = eh.at[1]
            @pl.when(sid < 10)
            def _(hh=hh):
                o = pl.multiple_of(sid * 1000, 8)
                pltpu.sync_copy(hh.at[pl.ds(o, 1000)], h_s.at[pl.ds(o, 1000)])
            _stage_idx(sh, idxs_v, base, extra)
            _stage_idx(dh, idxd_v, base, extra)
            plsc.subcore_barrier()

            def gstart(r, b):
                pltpu.async_copy(h_s.at[idxs_v.at[r]], rows[b], gsem[b])

            def gwait(r, b):
                pltpu.make_async_copy(
                    h_s.at[idxs_v.at[r]], rows[b], gsem[b]).wait()

            def sstart(r, b):
                pltpu.async_copy(rows[b], agg.at[idxd_v.at[r]],
                                 ssem[b], add=True)

            def swait(r, b):
                pltpu.make_async_copy(
                    rows[b], agg.at[idxd_v.at[r]], ssem[b]).wait()

            for b in range(nbuf):
                gstart(b, b)

            def body(jj, _, gstart=gstart, gwait=gwait,
                     sstart=sstart, swait=swait, nrows=nrows):
                r0 = jj * nbuf
                for b in range(nbuf):
                    gwait(r0 + b, b)
                    sstart(r0 + b, b)
                for b in range(nbuf):
                    @pl.when(r0 + b + nbuf < nrows)
                    def _(b=b, r0=r0):
                        swait(r0 + b, b)
                        gstart(r0 + b + nbuf, b)
                return 0
            lax.fori_loop(0, RPT // nbuf, body, 0)
            @pl.when(extra)
            def _():
                gwait(RPT, 0)
                sstart(RPT, 0)
                swait(RPT, 0)

            @pl.when(jnp.logical_not(extra))
            def _():
                swait(RPT - nbuf, 0)
            for b in range(1, nbuf):
                swait(RPT - nbuf + b, b)
            plsc.subcore_barrier()

            @pl.when(sid < 10)
            def _(g=g):
                o = pl.multiple_of(sid * 1000, 8)
                pltpu.sync_copy(agg.at[pl.ds(o, 1000)],
                                out_h.at[g, cid, pl.ds(o, 1000)])
                if g == 0:
                    pltpu.sync_copy(zagg_h.at[pl.ds(o, 1000)],
                                    agg.at[pl.ds(o, 1000)])
            plsc.subcore_barrier()

    return k(h0, h1, e0, e1, zagg)


def _tc_norms(x0, x1, dp):
    def body(x0_ref, x1_ref, dp_ref, h0_ref, h1_ref, nd_ref):
        for g, (x_ref, h_ref) in enumerate(((x0_ref, h0_ref),
                                            (x1_ref, h1_ref))):
            od = dp_ref[0, 2 * g:2 * g + 1, :] + dp_ref[1, 2 * g:2 * g + 1, :]
            oc = jnp.transpose(lax.rsqrt(jnp.maximum(od, 1.0)), (1, 0))
            h_ref[...] = x_ref[...] * oc
            idg = (dp_ref[0, 2 * g + 1:2 * g + 2, :]
                   + dp_ref[1, 2 * g + 1:2 * g + 2, :])
            nd_ref[:, g:g + 1] = jnp.transpose(
                lax.rsqrt(jnp.maximum(idg, 1.0)), (1, 0))

    return pl.pallas_call(
        body,
        out_shape=(jax.ShapeDtypeStruct((N, F), jnp.float32),
                   jax.ShapeDtypeStruct((N, F), jnp.float32),
                   jax.ShapeDtypeStruct((N, 2), jnp.float32)),
    )(x0, x1, dp)


def _tc_final(ap, nd, W, b2):
    def body(ap_ref, nd_ref, w_ref, b_ref, out_ref):
        g = pl.program_id(0)
        agg = ap_ref[0, 0] + ap_ref[0, 1]
        norm = jnp.where(g == 0, nd_ref[:, 0:1], nd_ref[:, 1:2])
        out_ref[0] = jnp.dot(agg * norm, w_ref[...],
                             preferred_element_type=jnp.float32) + b_ref[...]

    return pl.pallas_call(
        body,
        grid=(2,),
        in_specs=[
            pl.BlockSpec((1, NC, N, F), lambda g: (g, 0, 0, 0)),
            pl.BlockSpec((N, 2), lambda g: (0, 0)),
            pl.BlockSpec((F, H), lambda g: (0, 0)),
            pl.BlockSpec((1, H), lambda g: (0, 0)),
        ],
        out_specs=pl.BlockSpec((1, N, H), lambda g: (g, 0, 0)),
        out_shape=jax.ShapeDtypeStruct((2, N, H), jnp.float32),
    )(ap, nd, W, b2)


def kernel(feats0, feats1, W, b, edge_index0, edge_index1):
    e0 = edge_index0.reshape(2, EROWS, LANE)
    e1 = edge_index1.reshape(2, EROWS, LANE)
    zdeg = jnp.zeros((N,), jnp.float32)
    zagg = jnp.zeros((N, F), jnp.float32)

    dp = _sc_degrees(e0, e1, zdeg)
    h0, h1, nd = _tc_norms(feats0, feats1, dp)
    ap = _sc_messages(h0, h1, e0, e1, zagg)
    out = _tc_final(ap, nd, W, b.reshape(1, H))
    return (out[0], out[1])


# speedup vs baseline: 1.1197x; 1.1197x over previous
import functools

import jax
import jax.numpy as jnp
from jax import lax
from jax.experimental import pallas as pl
from jax.experimental.pallas import tpu as pltpu
from jax.experimental.pallas import tpu_sc as plsc

N = 10000
E = 320000
F = 64
H = 128
NC = 2
NS = 16
NW = NC * NS

LANE = 128
EROWS = E // LANE
RPT = EROWS // NW
REM = EROWS - RPT * NW


def _wid_rows(cid, sid):
    wid = cid * NS + sid
    base = wid * RPT + jnp.minimum(wid, REM)
    extra = wid < REM
    return base, extra


def _stage_idx(eh, buf, base, extra):
    pltpu.sync_copy(eh.at[pl.ds(base, RPT)], buf.at[pl.ds(0, RPT)])

    @pl.when(extra)
    def _():
        pltpu.sync_copy(eh.at[pl.ds(base + RPT, 1)], buf.at[pl.ds(RPT, 1)])


def _sc_degrees(e0, e1, zdeg):
    mesh = plsc.VectorSubcoreMesh(core_axis_name="c", subcore_axis_name="s")

    @functools.partial(
        pl.kernel,
        out_type=jax.ShapeDtypeStruct((NC, 4, N), jnp.float32),
        mesh=mesh,
        scratch_types=(
            [pltpu.VMEM_SHARED((N,), jnp.float32)] * 4
            + [pltpu.VMEM((RPT + 1, LANE), jnp.int32)] * 4
            + [pltpu.VMEM((LANE,), jnp.float32)]
            + [pltpu.SemaphoreType.DMA] * 4
        ),
        compiler_params=pltpu.CompilerParams(use_tc_tiling_on_sc=False),
    )
    def k(e0_h, e1_h, zdeg_h, out_h, *refs):
        accs = refs[0:4]
        idxs = refs[4:8]
        ones_v = refs[8]
        sems = refs[9:13]
        cid = lax.axis_index("c")
        sid = lax.axis_index("s")
        base, extra = _wid_rows(cid, sid)

        @pl.when(sid == 0)
        def _():
            for acc in accs:
                pltpu.sync_copy(zdeg_h, acc)

        for i in range(LANE // 16):
            ones_v[pl.ds(i * 16, 16)] = jnp.full((16,), 1.0, jnp.float32)
        for eh, buf in zip((e0_h.at[0], e0_h.at[1], e1_h.at[0], e1_h.at[1]),
                           idxs):
            _stage_idx(eh, buf, base, extra)
        plsc.subcore_barrier()

        def sstart(r, k):
            pltpu.async_copy(ones_v, accs[k].at[idxs[k].at[r]],
                             sems[k], add=True)

        def swait(r, k):
            pltpu.make_async_copy(ones_v, accs[k].at[idxs[k].at[r]],
                                  sems[k]).wait()

        for k4 in range(4):
            sstart(0, k4)

        def body(r, _):
            for k4 in range(4):
                swait(r - 1, k4)
                sstart(r, k4)
            return 0
        lax.fori_loop(1, RPT, body, 0)
        for k4 in range(4):
            swait(RPT - 1, k4)

        @pl.when(extra)
        def _():
            for k4 in range(4):
                sstart(RPT, k4)
            for k4 in range(4):
                swait(RPT, k4)
        plsc.subcore_barrier()

        @pl.when(sid < 10)
        def _():
            o = pl.multiple_of(sid * 1000, 8)
            for j, acc in enumerate(accs):
                pltpu.sync_copy(acc.at[pl.ds(o, 1000)],
                                out_h.at[cid, j, pl.ds(o, 1000)])

    return k(e0, e1, zdeg)


def _sc_messages(h0, h1, e0, e1, zagg):
    mesh = plsc.VectorSubcoreMesh(core_axis_name="c", subcore_axis_name="s")
    nbuf = 2

    @functools.partial(
        pl.kernel,
        out_type=jax.ShapeDtypeStruct((2, NC, N, F), jnp.float32),
        mesh=mesh,
        scratch_types=(
            [pltpu.VMEM_SHARED((N, F), jnp.float32),
             pltpu.VMEM_SHARED((N, F), jnp.float32),
             pltpu.VMEM((RPT + 1, LANE), jnp.int32),
             pltpu.VMEM((RPT + 1, LANE), jnp.int32)]
            + [pltpu.VMEM((LANE, F), jnp.float32)] * nbuf
            + [pltpu.SemaphoreType.DMA] * (2 * nbuf)
        ),
        compiler_params=pltpu.CompilerParams(use_tc_tiling_on_sc=False),
    )
    def k(h0_h, h1_h, e0_h, e1_h, zagg_h, out_h,
          agg, h_s, idxs_v, idxd_v, *bufs_and_sems):
        rows = bufs_and_sems[:nbuf]
        gsem = bufs_and_sems[nbuf:2 * nbuf]
        ssem = bufs_and_sems[2 * nbuf:]
        cid = lax.axis_index("c")
        sid = lax.axis_index("s")
        base, extra = _wid_rows(cid, sid)
        nrows = RPT + extra.astype(jnp.int32)

        @pl.when(sid == 0)
        def _():
            pltpu.sync_copy(zagg_h, agg)

        for g, (eh, hh) in enumerate(((e0_h, h0_h), (e1_h, h1_h))):
            sh = eh.at[0]
            dh = eh.at[1]
            @pl.when(sid < 10)
            def _(hh=hh):
                o = pl.multiple_of(sid * 1000, 8)
                pltpu.sync_copy(hh.at[pl.ds(o, 1000)], h_s.at[pl.ds(o, 1000)])
            _stage_idx(sh, idxs_v, base, extra)
            _stage_idx(dh, idxd_v, base, extra)
            plsc.subcore_barrier()

            def gstart(r, b):
                pltpu.async_copy(h_s.at[idxs_v.at[r]], rows[b], gsem[b])

            def gwait(r, b):
                pltpu.make_async_copy(
                    h_s.at[idxs_v.at[r]], rows[b], gsem[b]).wait()

            def sstart(r, b):
                pltpu.async_copy(rows[b], agg.at[idxd_v.at[r]],
                                 ssem[b], add=True)

            def swait(r, b):
                pltpu.make_async_copy(
                    rows[b], agg.at[idxd_v.at[r]], ssem[b]).wait()

            for b in range(nbuf):
                gstart(b, b)

            def body(jj, _, gstart=gstart, gwait=gwait,
                     sstart=sstart, swait=swait, nrows=nrows):
                r0 = jj * nbuf
                for b in range(nbuf):
                    gwait(r0 + b, b)
                    sstart(r0 + b, b)
                for b in range(nbuf):
                    @pl.when(r0 + b + nbuf < nrows)
                    def _(b=b, r0=r0):
                        swait(r0 + b, b)
                        gstart(r0 + b + nbuf, b)
                return 0
            lax.fori_loop(0, RPT // nbuf, body, 0)
            @pl.when(extra)
            def _():
                gwait(RPT, 0)
                sstart(RPT, 0)
                swait(RPT, 0)

            @pl.when(jnp.logical_not(extra))
            def _():
                swait(RPT - nbuf, 0)
            for b in range(1, nbuf):
                swait(RPT - nbuf + b, b)
            plsc.subcore_barrier()

            @pl.when(sid < 10)
            def _(g=g):
                o = pl.multiple_of(sid * 1000, 8)
                pltpu.sync_copy(agg.at[pl.ds(o, 1000)],
                                out_h.at[g, cid, pl.ds(o, 1000)])
                if g == 0:
                    pltpu.sync_copy(zagg_h.at[pl.ds(o, 1000)],
                                    agg.at[pl.ds(o, 1000)])
            plsc.subcore_barrier()

    return k(h0, h1, e0, e1, zagg)


def _tc_norms(x0, x1, dp):
    def body(x0_ref, x1_ref, dp_ref, h0_ref, h1_ref, nd_ref):
        for g, (x_ref, h_ref) in enumerate(((x0_ref, h0_ref),
                                            (x1_ref, h1_ref))):
            od = dp_ref[0, 2 * g:2 * g + 1, :] + dp_ref[1, 2 * g:2 * g + 1, :]
            oc = jnp.transpose(lax.rsqrt(jnp.maximum(od, 1.0)), (1, 0))
            h_ref[...] = x_ref[...] * oc
            idg = (dp_ref[0, 2 * g + 1:2 * g + 2, :]
                   + dp_ref[1, 2 * g + 1:2 * g + 2, :])
            nd_ref[:, g:g + 1] = jnp.transpose(
                lax.rsqrt(jnp.maximum(idg, 1.0)), (1, 0))

    return pl.pallas_call(
        body,
        out_shape=(jax.ShapeDtypeStruct((N, F), jnp.float32),
                   jax.ShapeDtypeStruct((N, F), jnp.float32),
                   jax.ShapeDtypeStruct((N, 2), jnp.float32)),
    )(x0, x1, dp)


def _tc_final(ap, nd, W, b2):
    def body(ap_ref, nd_ref, w_ref, b_ref, out_ref):
        g = pl.program_id(0)
        agg = ap_ref[0, 0] + ap_ref[0, 1]
        norm = jnp.where(g == 0, nd_ref[:, 0:1], nd_ref[:, 1:2])
        out_ref[0] = jnp.dot(agg * norm, w_ref[...],
                             preferred_element_type=jnp.float32) + b_ref[...]

    return pl.pallas_call(
        body,
        grid=(2,),
        in_specs=[
            pl.BlockSpec((1, NC, N, F), lambda g: (g, 0, 0, 0)),
            pl.BlockSpec((N, 2), lambda g: (0, 0)),
            pl.BlockSpec((F, H), lambda g: (0, 0)),
            pl.BlockSpec((1, H), lambda g: (0, 0)),
        ],
        out_specs=pl.BlockSpec((1, N, H), lambda g: (g, 0, 0)),
        out_shape=jax.ShapeDtypeStruct((2, N, H), jnp.float32),
    )(ap, nd, W, b2)


def kernel(feats0, feats1, W, b, edge_index0, edge_index1):
    e0 = edge_index0.reshape(2, EROWS, LANE)
    e1 = edge_index1.reshape(2, EROWS, LANE)
    zdeg = jnp.zeros((N,), jnp.float32)
    zagg = jnp.zeros((N, F), jnp.float32)

    dp = _sc_degrees(e0, e1, zdeg)
    h0, h1, nd = _tc_norms(feats0, feats1, dp)
    ap = _sc_messages(h0, h1, e0, e1, zagg)
    out = _tc_final(ap, nd, W, b.reshape(1, H))
    return (out[0], out[1])


# speedup vs baseline: 1.1466x; 1.0239x over previous
import functools

import jax
import jax.numpy as jnp
from jax import lax
from jax.experimental import pallas as pl
from jax.experimental.pallas import tpu as pltpu
from jax.experimental.pallas import tpu_sc as plsc

N = 10000
E = 320000
F = 64
H = 128
NC = 2
NS = 16
NW = NC * NS

LANE = 128
EROWS = E // LANE
RPT = EROWS // NW
REM = EROWS - RPT * NW


def _wid_rows(cid, sid):
    wid = cid * NS + sid
    base = wid * RPT + jnp.minimum(wid, REM)
    extra = wid < REM
    return base, extra


def _stage_idx(eh, buf, base, extra):
    pltpu.sync_copy(eh.at[pl.ds(base, RPT)], buf.at[pl.ds(0, RPT)])

    @pl.when(extra)
    def _():
        pltpu.sync_copy(eh.at[pl.ds(base + RPT, 1)], buf.at[pl.ds(RPT, 1)])


def _sc_degrees(e0, e1, zdeg):
    mesh = plsc.VectorSubcoreMesh(core_axis_name="c", subcore_axis_name="s")

    @functools.partial(
        pl.kernel,
        out_type=jax.ShapeDtypeStruct((NC, 4, N), jnp.float32),
        mesh=mesh,
        scratch_types=(
            [pltpu.VMEM_SHARED((N,), jnp.float32)] * 4
            + [pltpu.VMEM((RPT + 1, LANE), jnp.int32)] * 4
            + [pltpu.VMEM((LANE,), jnp.float32)]
            + [pltpu.SemaphoreType.DMA] * 4
        ),
        compiler_params=pltpu.CompilerParams(use_tc_tiling_on_sc=False),
    )
    def k(e0_h, e1_h, zdeg_h, out_h, *refs):
        accs = refs[0:4]
        idxs = refs[4:8]
        ones_v = refs[8]
        sems = refs[9:13]
        cid = lax.axis_index("c")
        sid = lax.axis_index("s")
        base, extra = _wid_rows(cid, sid)

        @pl.when(sid == 0)
        def _():
            for acc in accs:
                pltpu.sync_copy(zdeg_h, acc)

        for i in range(LANE // 16):
            ones_v[pl.ds(i * 16, 16)] = jnp.full((16,), 1.0, jnp.float32)
        for eh, buf in zip((e0_h.at[0], e0_h.at[1], e1_h.at[0], e1_h.at[1]),
                           idxs):
            _stage_idx(eh, buf, base, extra)
        plsc.subcore_barrier()

        def sstart(r, k):
            pltpu.async_copy(ones_v, accs[k].at[idxs[k].at[r]],
                             sems[k], add=True)

        def swait(r, k):
            pltpu.make_async_copy(ones_v, accs[k].at[idxs[k].at[r]],
                                  sems[k]).wait()

        for k4 in range(4):
            sstart(0, k4)

        def body(r, _):
            for k4 in range(4):
                swait(r - 1, k4)
                sstart(r, k4)
            return 0
        lax.fori_loop(1, RPT, body, 0)
        for k4 in range(4):
            swait(RPT - 1, k4)

        @pl.when(extra)
        def _():
            for k4 in range(4):
                sstart(RPT, k4)
            for k4 in range(4):
                swait(RPT, k4)
        plsc.subcore_barrier()

        @pl.when(sid < 10)
        def _():
            o = pl.multiple_of(sid * 1000, 8)
            for j, acc in enumerate(accs):
                pltpu.sync_copy(acc.at[pl.ds(o, 1000)],
                                out_h.at[cid, j, pl.ds(o, 1000)])

    return k(e0, e1, zdeg)


def _sc_messages(h0, h1, e0, e1, zagg):
    mesh = plsc.VectorSubcoreMesh(core_axis_name="c", subcore_axis_name="s")
    nbuf = 2

    @functools.partial(
        pl.kernel,
        out_type=jax.ShapeDtypeStruct((2, NC, N, F), jnp.float32),
        mesh=mesh,
        scratch_types=(
            [pltpu.VMEM_SHARED((N, F), jnp.float32),
             pltpu.VMEM_SHARED((N, F), jnp.float32),
             pltpu.VMEM((RPT + 1, LANE), jnp.int32),
             pltpu.VMEM((RPT + 1, LANE), jnp.int32)]
            + [pltpu.VMEM((LANE, F), jnp.float32)] * nbuf
            + [pltpu.SemaphoreType.DMA] * (2 * nbuf)
        ),
        compiler_params=pltpu.CompilerParams(use_tc_tiling_on_sc=False),
    )
    def k(h0_h, h1_h, e0_h, e1_h, zagg_h, out_h,
          agg, h_s, idxs_v, idxd_v, *bufs_and_sems):
        rows = bufs_and_sems[:nbuf]
        gsem = bufs_and_sems[nbuf:2 * nbuf]
        ssem = bufs_and_sems[2 * nbuf:]
        cid = lax.axis_index("c")
        sid = lax.axis_index("s")
        base, extra = _wid_rows(cid, sid)
        nrows = RPT + extra.astype(jnp.int32)

        @pl.when(sid == 0)
        def _():
            pltpu.sync_copy(zagg_h, agg)

        for g, (eh, hh) in enumerate(((e0_h, h0_h), (e1_h, h1_h))):
            sh = eh.at[0]
            dh = eh.at[1]
            @pl.when(sid < 10)
            def _(hh=hh):
                o = pl.multiple_of(sid * 1000, 8)
                pltpu.sync_copy(hh.at[pl.ds(o, 1000)], h_s.at[pl.ds(o, 1000)])
            _stage_idx(sh, idxs_v, base, extra)
            _stage_idx(dh, idxd_v, base, extra)
            plsc.subcore_barrier()

            def gstart(r, b):
                pltpu.async_copy(h_s.at[idxs_v.at[r]], rows[b], gsem[b])

            def gwait(r, b):
                pltpu.make_async_copy(
                    h_s.at[idxs_v.at[r]], rows[b], gsem[b]).wait()

            def sstart(r, b):
                pltpu.async_copy(rows[b], agg.at[idxd_v.at[r]],
                                 ssem[b], add=True)

            def swait(r, b):
                pltpu.make_async_copy(
                    rows[b], agg.at[idxd_v.at[r]], ssem[b]).wait()

            for b in range(nbuf):
                gstart(b, b)

            def body(jj, _, gstart=gstart, gwait=gwait,
                     sstart=sstart, swait=swait, nrows=nrows):
                r0 = jj * nbuf
                for b in range(nbuf):
                    gwait(r0 + b, b)
                    sstart(r0 + b, b)
                for b in range(nbuf):
                    @pl.when(r0 + b + nbuf < nrows)
                    def _(b=b, r0=r0):
                        swait(r0 + b, b)
                        gstart(r0 + b + nbuf, b)
                return 0
            lax.fori_loop(0, RPT // nbuf, body, 0)
            @pl.when(extra)
            def _():
                gwait(RPT, 0)
                sstart(RPT, 0)
                swait(RPT, 0)

            @pl.when(jnp.logical_not(extra))
            def _():
                swait(RPT - nbuf, 0)
            for b in range(1, nbuf):
                swait(RPT - nbuf + b, b)
            plsc.subcore_barrier()

            @pl.when(sid < 10)
            def _(g=g):
                o = pl.multiple_of(sid * 1000, 8)
                pltpu.sync_copy(agg.at[pl.ds(o, 1000)],
                                out_h.at[g, cid, pl.ds(o, 1000)])
                if g == 0:
                    pltpu.sync_copy(zagg_h.at[pl.ds(o, 1000)],
                                    agg.at[pl.ds(o, 1000)])
            plsc.subcore_barrier()

    return k(h0, h1, e0, e1, zagg)


def _tc_norms(x0, x1, dp):
    def body(x0_ref, x1_ref, dp_ref, h0_ref, h1_ref, nd_ref):
        for g, (x_ref, h_ref) in enumerate(((x0_ref, h0_ref),
                                            (x1_ref, h1_ref))):
            od = dp_ref[0, 2 * g:2 * g + 1, :] + dp_ref[1, 2 * g:2 * g + 1, :]
            oc = jnp.transpose(lax.rsqrt(jnp.maximum(od, 1.0)), (1, 0))
            h_ref[...] = x_ref[...] * oc
            idg = (dp_ref[0, 2 * g + 1:2 * g + 2, :]
                   + dp_ref[1, 2 * g + 1:2 * g + 2, :])
            nd_ref[:, g:g + 1] = jnp.transpose(
                lax.rsqrt(jnp.maximum(idg, 1.0)), (1, 0))

    return pl.pallas_call(
        body,
        out_shape=(jax.ShapeDtypeStruct((N, F), jnp.float32),
                   jax.ShapeDtypeStruct((N, F), jnp.float32),
                   jax.ShapeDtypeStruct((N, 2), jnp.float32)),
    )(x0, x1, dp)


def _tc_final(ap, nd, W, b2):
    def body(ap_ref, nd_ref, w_ref, b_ref, o0_ref, o1_ref):
        w = w_ref[...]
        bb = b_ref[...]
        for g, o_ref in enumerate((o0_ref, o1_ref)):
            agg = ap_ref[g, 0] + ap_ref[g, 1]
            norm = nd_ref[:, g:g + 1]
            o_ref[...] = jnp.dot(agg * norm, w,
                                 preferred_element_type=jnp.float32) + bb

    return pl.pallas_call(
        body,
        out_shape=(jax.ShapeDtypeStruct((N, H), jnp.float32),
                   jax.ShapeDtypeStruct((N, H), jnp.float32)),
    )(ap, nd, W, b2)


def kernel(feats0, feats1, W, b, edge_index0, edge_index1):
    e0 = edge_index0.reshape(2, EROWS, LANE)
    e1 = edge_index1.reshape(2, EROWS, LANE)
    zdeg = jnp.zeros((N,), jnp.float32)
    zagg = jnp.zeros((N, F), jnp.float32)

    dp = _sc_degrees(e0, e1, zdeg)
    h0, h1, nd = _tc_norms(feats0, feats1, dp)
    ap = _sc_messages(h0, h1, e0, e1, zagg)
    return _tc_final(ap, nd, W, b.reshape(1, H))
